# Initial kernel scaffold; baseline (speedup 1.0000x reference)
import jax
import jax.numpy as jnp
from jax.experimental import pallas as pl


def kernel(x, edge_index, edge_weight, W1, b1, gamma1, beta1, W2, b2, gamma2, beta2):
    raise NotImplementedError("write your pallas kernel here")


# speedup vs baseline: 3.0843x; 3.0843x over previous
import functools

import jax
import jax.numpy as jnp
from jax import lax
from jax.experimental import pallas as pl
from jax.experimental.pallas import tpu as pltpu
from jax.experimental.pallas import tpu_sc as plsc

_C = 128
_NT = 16
_NC = 2
_RC = 80
_W = 128


def _lmul_partial(z, src, dst, w, dused):
    V, width = z.shape
    epad = src.shape[0]
    nchunks = epad // (_NC * _NT * _C)
    nrc = V // _RC
    rc_per_tile = -(-nrc // _NT)
    mesh = plsc.VectorSubcoreMesh(core_axis_name="c", subcore_axis_name="s")

    @functools.partial(
        pl.kernel,
        mesh=mesh,
        out_type=[
            jax.ShapeDtypeStruct((V, width), jnp.float32),
            jax.ShapeDtypeStruct((V, width), jnp.float32),
        ],
        scratch_types=[
            pltpu.VMEM_SHARED((V, width), jnp.float32),
            pltpu.VMEM((_C,), jnp.int32),
            pltpu.VMEM((_C,), jnp.int32),
            pltpu.VMEM((_C,), jnp.float32),
            pltpu.VMEM((_C, width), jnp.float32),
            pltpu.VMEM((_RC, width), jnp.float32),
            pltpu.SemaphoreType.DMA,
        ],
    )
    def k(z_h, src_h, dst_h, w_h, o0_h, o1_h,
          acc, src_v, dst_v, w_v, rows, stage, sem):
        c = lax.axis_index("c")
        s = lax.axis_index("s")

        def zbody(i, carry):
            for j in range(width // 16):
                stage[i, pl.ds(j * 16, 16)] = jnp.zeros((16,), jnp.float32)
            return carry
        lax.fori_loop(0, _RC, zbody, 0)
        for t in range(rc_per_tile):
            cidx = s + t * _NT

            @pl.when(cidx < nrc)
            def _():
                pltpu.sync_copy(stage, acc.at[pl.ds(cidx * _RC, _RC)])
        plsc.subcore_barrier()

        base = (c * _NT + s) * (nchunks * _C)
        def chunk(kk, carry):
            off = base + kk * _C
            pltpu.sync_copy(src_h.at[pl.ds(off, _C)], src_v)
            pltpu.sync_copy(dst_h.at[pl.ds(off, _C)], dst_v)
            pltpu.sync_copy(w_h.at[pl.ds(off, _C)], w_v)
            pltpu.async_copy(z_h.at[src_v], rows, sem).wait()

            def scale(g, carry2):
                w16 = w_v[pl.ds(g * 16, 16)]
                for e in range(16):
                    we = w16[e]
                    row = g * 16 + e
                    for j in range(dused // 16):
                        sl = pl.ds(j * 16, 16)
                        rows[row, sl] = rows[row, sl] * we
                return carry2
            lax.fori_loop(0, _C // 16, scale, 0)

            pltpu.sync_copy(rows, acc.at[dst_v], add=True)
            return carry
        lax.fori_loop(0, nchunks, chunk, 0)
        plsc.subcore_barrier()

        for t in range(rc_per_tile):
            cidx = s + t * _NT

            @pl.when(cidx < nrc)
            def _():
                rr = cidx * _RC
                pltpu.sync_copy(acc.at[pl.ds(rr, _RC)], stage)

                @pl.when(c == 0)
                def _():
                    pltpu.sync_copy(stage, o0_h.at[pl.ds(rr, _RC)])

                @pl.when(c == 1)
                def _():
                    pltpu.sync_copy(stage, o1_h.at[pl.ds(rr, _RC)])

    return k(z, src, dst, w)


def _dense_layer(t0, u1, u2, bw0, bw1, bw2, b_cols, gam, bet,
                 smat, smat_t, n_copies):
    V = t0.shape[0]
    dout = bw0.shape[1]
    cnt = float(n_copies * V)

    def body(t0_r, u1_r, u2_r, bw0_r, bw1_r, bw2_r,
             bc_r, g_r, be_r, s_r, st_r, o_r):
        h = (jnp.dot(t0_r[...], bw0_r[...], preferred_element_type=jnp.float32)
             + jnp.dot(u1_r[...], bw1_r[...], preferred_element_type=jnp.float32)
             + jnp.dot(u2_r[...], bw2_r[...], preferred_element_type=jnp.float32)
             + bc_r[...])
        smat_v = s_r[...]
        mean_o = jnp.dot(jnp.sum(h, axis=0, keepdims=True), smat_v) / cnt
        ms_o = jnp.dot(jnp.sum(h * h, axis=0, keepdims=True), smat_v) / cnt
        var_o = ms_o - mean_o * mean_o
        scale_o = g_r[...] / jnp.sqrt(var_o + 1e-5)
        shift_o = be_r[...] - mean_o * scale_o
        scale_c = jnp.dot(scale_o, st_r[...])
        shift_c = jnp.dot(shift_o, st_r[...])
        o_r[...] = jnp.maximum(h * scale_c + shift_c, 0.0)

    return pl.pallas_call(
        body,
        out_shape=jax.ShapeDtypeStruct((V, dout), jnp.float32),
    )(t0, u1, u2, bw0, bw1, bw2, b_cols, gam, bet, smat, smat_t)


def kernel(x, edge_index, edge_weight, W1, b1, gamma1, beta1,
           W2, b2, gamma2, beta2):
    B, Cin, V, X, Y, Z = x.shape
    N = B * X * Y * Z
    E = edge_index.shape[1]

    xt = jnp.transpose(x, (0, 3, 4, 5, 2, 1)).reshape(N, V, Cin)
    xt = jnp.transpose(xt, (1, 0, 2)).reshape(V, N * Cin)
    d1 = N * Cin
    xt = jnp.pad(xt, ((0, 0), (0, _W - d1)))

    nchunks = -(-E // (_NC * _NT * _C))
    epad = nchunks * _NC * _NT * _C
    pad = epad - E
    src = jnp.concatenate(
        [edge_index[0].astype(jnp.int32), jnp.zeros((pad,), jnp.int32)])
    dst = jnp.concatenate(
        [edge_index[1].astype(jnp.int32), jnp.zeros((pad,), jnp.int32)])
    w = jnp.concatenate([edge_weight, jnp.zeros((pad,), jnp.float32)])

    eye_n = jnp.eye(N, dtype=jnp.float32)

    def layer(h_in, dused, Wk, bias, gam, bet):
        fout = Wk.shape[2]
        p0, p1 = _lmul_partial(h_in, src, dst, w, dused)
        u1 = p0 + p1
        q0, q1 = _lmul_partial(u1, src, dst, w, dused)
        u2 = q0 + q1
        din = dused // N
        bw0 = jnp.kron(eye_n, Wk[0] - Wk[2])
        bw1 = jnp.kron(eye_n, Wk[1])
        bw2 = jnp.kron(eye_n, 2.0 * Wk[2])
        zpad = ((0, _W - N * din), (0, 0))
        bw0, bw1, bw2 = (jnp.pad(m, zpad) for m in (bw0, bw1, bw2))
        b_cols = jnp.tile(bias, N)[None]
        smat = jnp.tile(jnp.eye(fout, dtype=jnp.float32), (N, 1))
        return _dense_layer(h_in, u1, u2, bw0, bw1, bw2,
                            b_cols, gam[None], bet[None], smat, smat.T, N)

    h1 = layer(xt, d1, W1, b1, gamma1, beta1)
    h2 = layer(h1, N * W2.shape[1], W2, b2, gamma2, beta2)

    cout = W2.shape[2]
    out = jnp.transpose(h2.reshape(V, B, X, Y, Z, cout), (1, 5, 0, 2, 3, 4))
    return out
